# Initial kernel scaffold; baseline (speedup 1.0000x reference)
import jax
import jax.numpy as jnp
from jax.experimental import pallas as pl


def kernel(x, edge_index, edge_attr, Wq1, bq1, Wk1, bk1, Wv1, bv1, We1, Ws1, bs1, Wb1, Wt1, bt1, g1, be1, Wq2, bq2, Wk2, bk2, Wv2, bv2, We2, Ws2, bs2, Wb2, Wt2, bt2, g2, be2):
    raise NotImplementedError("write your pallas kernel here")


# speedup vs baseline: 1.1977x; 1.1977x over previous
import functools

import jax
import jax.numpy as jnp
from jax.experimental import pallas as pl
from jax.experimental.pallas import tpu as pltpu

N = 10000
E = 320000
H = 4


def _post_l2_body(out_ref, xr_ref, wb_ref, wt_ref, bt_ref, g_ref, be_ref, y_ref):
    out = out_ref[...]
    xr = xr_ref[...]
    wb = wb_ref[...]
    wa = wb[:, 0:4]
    wbb = wb[:, 4:8]
    wc = wb[:, 8:12]
    lin = (jnp.sum(out * wa, axis=1, keepdims=True)
           + jnp.sum(xr * wbb, axis=1, keepdims=True)
           + jnp.sum((out - xr) * wc, axis=1, keepdims=True))
    beta = jax.nn.sigmoid(lin)
    h = beta * xr + (1.0 - beta) * out
    y = jnp.sum(h * wt_ref[...], axis=1, keepdims=True) + bt_ref[0, 0]
    mu = jnp.mean(y)
    var = jnp.mean(jnp.square(y - mu))
    y_ref[...] = (y - mu) / jnp.sqrt(var + 1e-5) * g_ref[0, 0] + be_ref[0, 0]


def _post_l2(out2, xr2, Wb2, Wt2, bt2, g2, be2):
    return pl.pallas_call(
        _post_l2_body,
        out_shape=jax.ShapeDtypeStruct((N, 1), jnp.float32),
    )(out2, xr2, Wb2.reshape(1, 12), Wt2.reshape(1, 4),
      bt2.reshape(1, 1), g2.reshape(1, 1), be2.reshape(1, 1))


def _tconv_edges(x_proj_q, x_proj_k, x_proj_v, e, src, dst, heads, C):
    m = src.shape[0]
    q = x_proj_q[dst].reshape(m, heads, C)
    k = x_proj_k[src].reshape(m, heads, C)
    v = x_proj_v[src].reshape(m, heads, C)
    alpha = jnp.sum(q * (k + e), axis=-1) / jnp.sqrt(float(C))
    ex = jnp.exp(alpha)
    den = jax.ops.segment_sum(ex, dst, num_segments=N)
    a = ex / (den[dst] + 1e-16)
    msg = (v + e) * a[..., None]
    return jax.ops.segment_sum(msg.reshape(m, heads * C), dst, num_segments=N)


def kernel(x, edge_index, edge_attr, Wq1, bq1, Wk1, bk1, Wv1, bv1, We1, Ws1,
           bs1, Wb1, Wt1, bt1, g1, be1, Wq2, bq2, Wk2, bk2, Wv2, bv2, We2,
           Ws2, bs2, Wb2, Wt2, bt2, g2, be2):
    src = edge_index[0]
    dst = edge_index[1]

    q1 = x @ Wq1 + bq1
    k1 = x @ Wk1 + bk1
    v1 = x @ Wv1 + bv1
    e1 = (edge_attr @ We1).reshape(E, H, 128)
    out1 = _tconv_edges(q1, k1, v1, e1, src, dst, H, 128)
    xr1 = x @ Ws1 + bs1
    beta1 = jax.nn.sigmoid(jnp.concatenate([out1, xr1, out1 - xr1], axis=-1) @ Wb1)
    h = beta1 * xr1 + (1.0 - beta1) * out1
    h = h @ Wt1 + bt1
    mu = h.mean(axis=0)
    var = h.var(axis=0)
    h = (h - mu) / jnp.sqrt(var + 1e-5) * g1 + be1

    q2 = h @ Wq2 + bq2
    k2 = h @ Wk2 + bk2
    v2 = h @ Wv2 + bv2
    e2 = (edge_attr @ We2).reshape(E, H, 1)
    out2 = _tconv_edges(q2, k2, v2, e2, src, dst, H, 1)
    xr2 = h @ Ws2 + bs2
    return _post_l2(out2, xr2, Wb2, Wt2, bt2, g2, be2)


# speedup vs baseline: 1.2464x; 1.0406x over previous
import functools

import jax
import jax.numpy as jnp
from jax import lax
from jax.experimental import pallas as pl
from jax.experimental.pallas import tpu as pltpu
from jax.experimental.pallas import tpu_sc as plsc

N = 10000
E = 320000
H = 4
C1 = 128

NC = 2
NS = 16
NW = NC * NS
L = 16

EPW = E // NW
BB = 80
NIT = EPW // BB


def _sc_mesh():
    return plsc.VectorSubcoreMesh(core_axis_name="c", subcore_axis_name="s",
                                  num_cores=NC, num_subcores=NS)


def _alpha1_body(qs_hbm, k_hbm, qwe_hbm, src_hbm, dst_hbm, ea_hbm, zer_hbm,
                 ex_hbm, denp_hbm,
                 src_v, dst_v, ea_v, qrows, krows, qwerows, exbuf, exT,
                 den_sh, sem):
    cid = lax.axis_index("c")
    sid = lax.axis_index("s")
    wid = sid * NC + cid

    rz = (N // NS) // 8 * 8
    rem = N - NS * rz
    rbase = sid * rz
    pltpu.sync_copy(zer_hbm.at[pl.ds(rbase, rz)],
                    den_sh.at[pl.ds(rbase, rz)])
    if rem:
        @pl.when(sid == 0)
        def _():
            pltpu.sync_copy(zer_hbm.at[pl.ds(NS * rz, rem)],
                            den_sh.at[pl.ds(NS * rz, rem)])
    plsc.subcore_barrier()

    def zrow(i, _z):
        exT[i, :] = jnp.zeros((16,), jnp.float32)
        return _z
    lax.fori_loop(0, BB, zrow, 0)

    def batch(it, _):
        base = wid * EPW + it * BB
        pltpu.sync_copy(src_hbm.at[pl.ds(base, BB)], src_v)
        pltpu.sync_copy(dst_hbm.at[pl.ds(base, BB)], dst_v)
        pltpu.sync_copy(ea_hbm.at[pl.ds(base, BB)], ea_v)
        pltpu.async_copy(qs_hbm.at[dst_v], qrows, sem).wait()
        pltpu.async_copy(k_hbm.at[src_v], krows, sem).wait()
        pltpu.async_copy(qwe_hbm.at[dst_v], qwerows, sem).wait()

        def group(g, _2):
            rowv = lax.iota(jnp.int32, L) + g * L
            eav = ea_v[pl.ds(g * L, L)]

            def head(h, _3):
                hv = jnp.full((L,), h, jnp.int32)

                def dot_c(c, acc):
                    colv = jnp.full((L,), h * C1 + c, jnp.int32)
                    qv = plsc.load_gather(qrows, [rowv, colv])
                    kv = plsc.load_gather(krows, [rowv, colv])
                    return acc + qv * kv

                acc = lax.fori_loop(0, C1, dot_c, jnp.zeros((L,), jnp.float32),
                                    unroll=4)
                qwev = plsc.load_gather(qwerows, [rowv, hv])
                ex = jnp.exp(acc + eav * qwev)
                exbuf[pl.ds(h * BB + g * L, L)] = ex
                plsc.store_scatter(exT, [rowv, hv], ex)
                return _3

            return lax.fori_loop(0, H, head, _2)

        lax.fori_loop(0, BB // L, group, 0)

        for h in range(H):
            pltpu.sync_copy(exbuf.at[pl.ds(h * BB, BB)],
                            ex_hbm.at[pl.ds(h * E + base, BB)])
        pltpu.sync_copy(exT, den_sh.at[dst_v], add=True)
        return _

    lax.fori_loop(0, NIT, batch, 0)

    plsc.subcore_barrier()
    pltpu.sync_copy(den_sh.at[pl.ds(rbase, rz)],
                    denp_hbm.at[cid, pl.ds(rbase, rz)])
    if rem:
        @pl.when(sid == 0)
        def _():
            pltpu.sync_copy(den_sh.at[pl.ds(NS * rz, rem)],
                            denp_hbm.at[cid, pl.ds(NS * rz, rem)])


def _alpha1(qs, k, qwe, src, dst, ea, zer):
    f = pl.kernel(
        _alpha1_body,
        out_type=[jax.ShapeDtypeStruct((H * E,), jnp.float32),
                  jax.ShapeDtypeStruct((NC, N, 16), jnp.float32)],
        mesh=_sc_mesh(),
        scratch_types=[
            pltpu.VMEM((BB,), jnp.int32),
            pltpu.VMEM((BB,), jnp.int32),
            pltpu.VMEM((BB,), jnp.float32),
            pltpu.VMEM((BB, 4 * C1), jnp.float32),
            pltpu.VMEM((BB, 4 * C1), jnp.float32),
            pltpu.VMEM((BB, 16), jnp.float32),
            pltpu.VMEM((H * BB,), jnp.float32),
            pltpu.VMEM((BB, 16), jnp.float32),
            pltpu.VMEM_SHARED((N, 16), jnp.float32),
            pltpu.SemaphoreType.DMA,
        ],
        compiler_params=pltpu.CompilerParams(use_tc_tiling_on_sc=False,
                                             needs_layout_passes=False),
    )
    return f(qs, k, qwe, src, dst, ea, zer)


def _post_l2_body(out_ref, xr_ref, wb_ref, wt_ref, bt_ref, g_ref, be_ref, y_ref):
    out = out_ref[...]
    xr = xr_ref[...]
    wb = wb_ref[...]
    wa = wb[:, 0:4]
    wbb = wb[:, 4:8]
    wc = wb[:, 8:12]
    lin = (jnp.sum(out * wa, axis=1, keepdims=True)
           + jnp.sum(xr * wbb, axis=1, keepdims=True)
           + jnp.sum((out - xr) * wc, axis=1, keepdims=True))
    beta = jax.nn.sigmoid(lin)
    h = beta * xr + (1.0 - beta) * out
    y = jnp.sum(h * wt_ref[...], axis=1, keepdims=True) + bt_ref[0, 0]
    mu = jnp.mean(y)
    var = jnp.mean(jnp.square(y - mu))
    y_ref[...] = (y - mu) / jnp.sqrt(var + 1e-5) * g_ref[0, 0] + be_ref[0, 0]


def _post_l2(out2, xr2, Wb2, Wt2, bt2, g2, be2):
    return pl.pallas_call(
        _post_l2_body,
        out_shape=jax.ShapeDtypeStruct((N, 1), jnp.float32),
    )(out2, xr2, Wb2.reshape(1, 12), Wt2.reshape(1, 4),
      bt2.reshape(1, 1), g2.reshape(1, 1), be2.reshape(1, 1))


def kernel(x, edge_index, edge_attr, Wq1, bq1, Wk1, bk1, Wv1, bv1, We1, Ws1,
           bs1, Wb1, Wt1, bt1, g1, be1, Wq2, bq2, Wk2, bk2, Wv2, bv2, We2,
           Ws2, bs2, Wb2, Wt2, bt2, g2, be2):
    src = edge_index[0]
    dst = edge_index[1]
    ea = edge_attr.reshape(E)
    zer = jnp.zeros((N, 16), jnp.float32)

    rsc = 1.0 / jnp.sqrt(128.0)
    qs1 = (x @ Wq1 + bq1) * rsc
    k1 = x @ Wk1 + bk1
    v1 = x @ Wv1 + bv1
    qwe1 = jnp.sum((qs1 * We1).reshape(N, H, C1), axis=-1)
    qwe1p = jnp.concatenate([qwe1, jnp.zeros((N, 12), jnp.float32)], axis=1)

    EX, DENP = _alpha1(qs1, k1, qwe1p, src, dst, ea, zer)
    EX = EX.reshape(H, E)
    den = (DENP[0] + DENP[1])[:, :H]
    a = EX.T / (den[dst] + 1e-16)

    e1 = (edge_attr @ We1).reshape(E, H, C1)
    msg = (v1[src].reshape(E, H, C1) + e1) * a[..., None]
    out1 = jax.ops.segment_sum(msg.reshape(E, H * C1), dst, num_segments=N)

    xr1 = x @ Ws1 + bs1
    beta1 = jax.nn.sigmoid(jnp.concatenate([out1, xr1, out1 - xr1], axis=-1) @ Wb1)
    h = beta1 * xr1 + (1.0 - beta1) * out1
    h = h @ Wt1 + bt1
    mu = h.mean(axis=0)
    var = h.var(axis=0)
    h = (h - mu) / jnp.sqrt(var + 1e-5) * g1 + be1

    q2 = h @ Wq2 + bq2
    k2 = h @ Wk2 + bk2
    v2 = h @ Wv2 + bv2
    e2 = (edge_attr @ We2).reshape(E, H, 1)
    m2 = q2[dst].reshape(E, H, 1) * (k2[src].reshape(E, H, 1) + e2)
    alpha2 = jnp.sum(m2, axis=-1)
    ex2 = jnp.exp(alpha2)
    den2 = jax.ops.segment_sum(ex2, dst, num_segments=N)
    a2 = ex2 / (den2[dst] + 1e-16)
    msg2 = (v2[src].reshape(E, H, 1) + e2) * a2[..., None]
    out2 = jax.ops.segment_sum(msg2.reshape(E, H), dst, num_segments=N)
    xr2 = h @ Ws2 + bs2
    return _post_l2(out2, xr2, Wb2, Wt2, bt2, g2, be2)
